# baseline (speedup 1.0000x reference)
import functools

import jax
import jax.numpy as jnp
from jax import lax
from jax.experimental import pallas as pl
from jax.experimental.pallas import tpu as pltpu
from jax.experimental.pallas import tpu_sc as plsc

NC = 2
NS = 16
CH = 128
BN = 2000


def _sc_scatter_make(N, D, NCH):
    n_tab = -(-(N + 1) // (NS * 16)) * (NS * 16)
    rows_z = n_tab // NS
    mesh = plsc.VectorSubcoreMesh(core_axis_name="c", subcore_axis_name="s",
                                  num_cores=NC, num_subcores=NS)

    G = 16
    assert NCH % G == 0
    NG = NCH // G

    @functools.partial(
        pl.kernel,
        out_type=jax.ShapeDtypeStruct((NC, n_tab, D), jnp.float32),
        mesh=mesh,
        scratch_types=[
            pltpu.VMEM((G, CH), jnp.int32),
            pltpu.VMEM((G, CH), jnp.int32),
            pltpu.VMEM((CH, D), jnp.float32),
            pltpu.VMEM((CH, D), jnp.float32),
            pltpu.VMEM((16, D), jnp.float32),
            pltpu.VMEM_SHARED((n_tab, D), jnp.float32),
            pltpu.SemaphoreType.DMA,
            pltpu.SemaphoreType.DMA,
        ],
    )
    def k(es_hbm, ed_hbm, t_hbm, out_hbm, is_v, id_v, gb0, gb1, zbuf, agg,
          sem0, sem1):
        c = lax.axis_index("c")
        s = lax.axis_index("s")
        for i in range(16):
            for j in range(D // 16):
                zbuf[i, pl.ds(j * 16, 16)] = jnp.zeros((16,), jnp.float32)

        base = (c * NS + s) * NCH

        def zrow(kk, carry):
            pltpu.sync_copy(zbuf, agg.at[pl.ds(s * rows_z + kk * 16, 16)])
            return carry
        lax.fori_loop(0, rows_z // 16, zrow, 0)
        plsc.subcore_barrier()

        gbs = (gb0, gb1)
        sems = (sem0, sem1)

        def group(grp, carry):
            pltpu.sync_copy(es_hbm.at[pl.ds(base + grp * G, G)], is_v)
            pltpu.sync_copy(ed_hbm.at[pl.ds(base + grp * G, G)], id_v)
            pend = pltpu.async_copy(t_hbm.at[is_v.at[0]], gbs[0], sems[0])
            for j in range(G):
                b = j % 2
                nxt = None
                if j + 1 < G:
                    nxt = pltpu.async_copy(t_hbm.at[is_v.at[j + 1]],
                                           gbs[1 - b], sems[1 - b])
                pend.wait()
                pltpu.sync_copy(gbs[b], agg.at[id_v.at[j]], add=True)
                pend = nxt
            return carry
        lax.fori_loop(0, NG, group, 0)
        plsc.subcore_barrier()

        pltpu.sync_copy(agg.at[pl.ds(s * rows_z, rows_z)],
                        out_hbm.at[c, pl.ds(s * rows_z, rows_z)])

    return k


def _sc_deg_make(N, D, NCH):
    n_tab = -(-(N + 1) // (NS * 16)) * (NS * 16)
    rows_z = n_tab // NS
    G = 16
    assert NCH % G == 0
    mesh = plsc.VectorSubcoreMesh(core_axis_name="c", subcore_axis_name="s",
                                  num_cores=NC, num_subcores=NS)

    @functools.partial(
        pl.kernel,
        out_type=jax.ShapeDtypeStruct((NC, n_tab, D), jnp.float32),
        mesh=mesh,
        scratch_types=[
            pltpu.VMEM((G, CH), jnp.int32),
            pltpu.VMEM((CH, D), jnp.float32),
            pltpu.VMEM((16, D), jnp.float32),
            pltpu.VMEM_SHARED((n_tab, D), jnp.float32),
        ],
    )
    def k(ed_hbm, out_hbm, id_v, obuf, zbuf, deg):
        c = lax.axis_index("c")
        s = lax.axis_index("s")
        one0 = jnp.where(lax.iota(jnp.int32, 16) == 0,
                         jnp.float32(1.0), jnp.float32(0.0))
        zero = jnp.zeros((16,), jnp.float32)
        for i in range(CH):
            for j in range(D // 16):
                obuf[i, pl.ds(j * 16, 16)] = one0 if j == 0 else zero
        for i in range(16):
            for j in range(D // 16):
                zbuf[i, pl.ds(j * 16, 16)] = zero

        def zrow(kk, carry):
            pltpu.sync_copy(zbuf, deg.at[pl.ds(s * rows_z + kk * 16, 16)])
            return carry
        lax.fori_loop(0, rows_z // 16, zrow, 0)
        plsc.subcore_barrier()
        base = (c * NS + s) * NCH

        def group(grp, carry):
            pltpu.sync_copy(ed_hbm.at[pl.ds(base + grp * G, G)], id_v)
            for j in range(G):
                pltpu.sync_copy(obuf, deg.at[id_v.at[j]], add=True)
            return carry
        lax.fori_loop(0, NCH // G, group, 0)
        plsc.subcore_barrier()
        pltpu.sync_copy(deg.at[pl.ds(s * rows_z, rows_z)],
                        out_hbm.at[c, pl.ds(s * rows_z, rows_z)])

    return k


def _dinv_of(d0, d1):
    return lax.rsqrt(jnp.sum(d0 + d1, axis=1, keepdims=True) + 1.0)


def _tc_first_body(d0, d1, x, w, o):
    dinv = _dinv_of(d0[...], d1[...])
    o[...] = dinv * jnp.dot(x[...], w[...], preferred_element_type=jnp.float32)


def _tc_mid_body(d0, d1, s0, s1, t, b, w, o):
    dinv = _dinv_of(d0[...], d1[...])
    h = jnp.maximum(dinv * (s0[...] + s1[...] + t[...]) + b[...], 0.0)
    o[...] = dinv * jnp.dot(h, w[...], preferred_element_type=jnp.float32)


def _tc_last_body(d0, d1, s0, s1, t, b, x, wc, bcp, o):
    dinv = _dinv_of(d0[...], d1[...])
    h = jnp.maximum(dinv * (s0[...] + s1[...] + t[...]) + b[...], 0.0)
    o[...] = jnp.dot(h + x[...], wc[...],
                     preferred_element_type=jnp.float32) + bcp[...]


def _half_spec(cols, half):
    return pl.BlockSpec((None, BN, cols), lambda i, _h=half: (_h, i, 0))


def _row_spec(cols):
    return pl.BlockSpec((BN, cols), lambda i: (i, 0))


def _full_spec(r, c):
    return pl.BlockSpec((r, c), lambda i: (0, 0))


def _tc_first(deg2, x, w, N, D):
    return pl.pallas_call(
        _tc_first_body,
        grid=(N // BN,),
        in_specs=[_half_spec(D, 0), _half_spec(D, 1),
                  _row_spec(D), _full_spec(D, D)],
        out_specs=_row_spec(D),
        out_shape=jax.ShapeDtypeStruct((N, D), jnp.float32),
    )(deg2, deg2, x, w)


def _tc_mid(deg2, s2, t, b, w, N, D):
    return pl.pallas_call(
        _tc_mid_body,
        grid=(N // BN,),
        in_specs=[_half_spec(D, 0), _half_spec(D, 1),
                  _half_spec(D, 0), _half_spec(D, 1),
                  _row_spec(D), _full_spec(1, D), _full_spec(D, D)],
        out_specs=_row_spec(D),
        out_shape=jax.ShapeDtypeStruct((N, D), jnp.float32),
    )(deg2, deg2, s2, s2, t, b, w)


def _tc_last(deg2, s2, t, b, x, wcp, bcp, N, D):
    return pl.pallas_call(
        _tc_last_body,
        grid=(N // BN,),
        in_specs=[_half_spec(D, 0), _half_spec(D, 1),
                  _half_spec(D, 0), _half_spec(D, 1),
                  _row_spec(D), _full_spec(1, D),
                  _row_spec(D), _full_spec(D, D), _full_spec(1, D)],
        out_specs=_row_spec(D),
        out_shape=jax.ShapeDtypeStruct((N, D), jnp.float32),
    )(deg2, deg2, s2, s2, t, b, x, wcp, bcp)


def kernel(x, edge_index, Ws, bs, Wc, bc):
    N, D = x.shape
    E = edge_index.shape[1]
    L = Ws.shape[0]
    OUT = Wc.shape[1]

    NCH = -(-E // (NC * NS * CH))
    NCH += (-NCH) % 4
    e_pad = NC * NS * CH * NCH
    es_chunks = jnp.concatenate(
        [edge_index[0], jnp.zeros((e_pad - E,), jnp.int32)]
    ).reshape(NC * NS * NCH, CH)
    ed_chunks = jnp.concatenate(
        [edge_index[1], jnp.full((e_pad - E,), N, jnp.int32)]
    ).reshape(NC * NS * NCH, CH)

    sc_deg = _sc_deg_make(N, D, NCH)
    sc_scatter = _sc_scatter_make(N, D, NCH)

    deg2 = sc_deg(ed_chunks)
    t = _tc_first(deg2, x, Ws[0], N, D)
    for i in range(1, L):
        s2 = sc_scatter(es_chunks, ed_chunks, t)
        t = _tc_mid(deg2, s2, t, bs[i - 1].reshape(1, D), Ws[i], N, D)
    s2 = sc_scatter(es_chunks, ed_chunks, t)

    wcp = jnp.zeros((D, D), jnp.float32).at[:, :OUT].set(Wc)
    bcp = jnp.zeros((1, D), jnp.float32).at[0, :OUT].set(bc)
    out_p = _tc_last(deg2, s2, t, bs[L - 1].reshape(1, D), x, wcp, bcp, N, D)
    return out_p[:, :OUT]
